# baseline (speedup 1.0000x reference)
import jax
import jax.numpy as jnp
from jax import lax
from jax.experimental import pallas as pl
from jax.experimental.pallas import tpu as pltpu
from jax.experimental.pallas import tpu_sc as plsc

_NTOK = 16384
_D = 64
_VOCAB = 1000000
_NC = 2
_NS = 16
_NW = _NC * _NS
_PER_W = _NTOK // _NW
_CHUNK = 128
_NCHUNK = _PER_W // _CHUNK
_LANES = 16
_HPAD = 1000064
_ZCH = 62592
_ZLAST = _HPAD - 15 * _ZCH
_C = 32256
_NMAIN = 31 * _C
_TAIL = _VOCAB - _NMAIN


def _hist_body(ids_hbm, zeros_hbm, out_hbm, idx_v, vals_v, zbuf_v, hist_sh):
    c = lax.axis_index("c")
    s = lax.axis_index("s")
    wid = s * _NC + c

    pltpu.sync_copy(ids_hbm.at[wid], idx_v)

    for ci in range(_CHUNK // _LANES):
        vals_v[pl.ds(ci * _LANES, _LANES)] = jnp.full((_LANES,), 1.0,
                                                      jnp.float32)

    pltpu.sync_copy(zeros_hbm, zbuf_v)

    @pl.when(s < _NS - 1)
    def _zmain():
        pltpu.sync_copy(zbuf_v, hist_sh.at[pl.ds(s * _ZCH, _ZCH)])

    @pl.when(s == _NS - 1)
    def _zlast():
        pltpu.sync_copy(zbuf_v.at[pl.ds(0, _ZLAST)],
                        hist_sh.at[pl.ds(15 * _ZCH, _ZLAST)])

    plsc.subcore_barrier()

    for k in range(_NCHUNK):
        pltpu.sync_copy(vals_v, hist_sh.at[idx_v.at[k]], add=True)
    plsc.subcore_barrier()

    @pl.when(s < _NS - 1)
    def _dmain():
        pltpu.sync_copy(hist_sh.at[pl.ds(s * _ZCH, _ZCH)], zbuf_v)
        pltpu.sync_copy(zbuf_v, out_hbm.at[c, pl.ds(s * _ZCH, _ZCH)])

    @pl.when(s == _NS - 1)
    def _dlast():
        zpart = zbuf_v.at[pl.ds(0, _ZLAST)]
        pltpu.sync_copy(hist_sh.at[pl.ds(15 * _ZCH, _ZLAST)], zpart)
        pltpu.sync_copy(zpart, out_hbm.at[c, pl.ds(15 * _ZCH, _ZLAST)])


def _scan_body(tbl_ref, cnt_ref, o_ref):
    i = pl.program_id(0)

    @pl.when(i == 0)
    def _init():
        o_ref[...] = jnp.zeros_like(o_ref)

    csum = cnt_ref[0, :] + cnt_ref[1, :]
    o_ref[...] += jnp.dot(
        tbl_ref[...], csum, preferred_element_type=jnp.float32
    )[None, :]


def _finish_body(main_ref, ctail_ref, ttail_ref, wt_ref, o_ref):
    ct = ctail_ref[0, :] + ctail_ref[1, :]
    tail = jnp.dot(ttail_ref[...], ct, preferred_element_type=jnp.float32)
    pooled = (main_ref[0, :] + tail) * (1.0 / _NTOK)
    o_ref[...] = jnp.dot(pooled[None, :], wt_ref[...],
                         preferred_element_type=jnp.float32)


@jax.jit
def _run(ids, emb_t, wt):
    mesh = plsc.VectorSubcoreMesh(core_axis_name="c", subcore_axis_name="s")
    counts = pl.kernel(
        _hist_body,
        out_type=jax.ShapeDtypeStruct((_NC, _HPAD), jnp.float32),
        mesh=mesh,
        scratch_types=[
            pltpu.VMEM((_NCHUNK, _CHUNK), jnp.int32),
            pltpu.VMEM((_CHUNK,), jnp.float32),
            pltpu.VMEM((_ZCH,), jnp.float32),
            pltpu.VMEM_SHARED((_HPAD,), jnp.float32),
        ],
        name="token_histogram_sc",
    )(ids, jnp.zeros((_ZCH,), jnp.float32))

    main = pl.pallas_call(
        _scan_body,
        grid=(_NMAIN // _C,),
        in_specs=[
            pl.BlockSpec((_D, _C), lambda i: (0, i)),
            pl.BlockSpec((_NC, _C), lambda i: (0, i)),
        ],
        out_specs=pl.BlockSpec((1, _D), lambda i: (0, 0)),
        out_shape=jax.ShapeDtypeStruct((1, _D), jnp.float32),
        name="table_scan_matvec_tc",
    )(emb_t, counts)

    ctail = lax.slice(counts, (0, _NMAIN), (_NC, _VOCAB))
    ttail = lax.slice(emb_t, (0, _NMAIN), (_D, _VOCAB))
    out = pl.pallas_call(
        _finish_body,
        out_shape=jax.ShapeDtypeStruct((1, _D), jnp.float32),
        name="finish_tc",
    )(main, ctail, ttail, wt)
    return out


def kernel(token_ids, embedding, W):
    ids = token_ids.astype(jnp.int32).reshape(_NW, _NCHUNK, _CHUNK)
    return _run(ids, embedding.T, W.T)


# speedup vs baseline: 1.0388x; 1.0388x over previous
import jax
import jax.numpy as jnp
from jax import lax
from jax.experimental import pallas as pl
from jax.experimental.pallas import tpu as pltpu
from jax.experimental.pallas import tpu_sc as plsc

_NTOK = 16384
_D = 64
_VOCAB = 1000000
_NC = 2
_NS = 16
_NW = _NC * _NS
_PER_W = _NTOK // _NW
_CHUNK = 128
_NCHUNK = _PER_W // _CHUNK
_LANES = 16
_HPAD = 1000064
_ZCH = 62592
_ZLAST = _HPAD - 15 * _ZCH
_C = 32256
_NMAIN = 31 * _C
_TAIL = _VOCAB - _NMAIN


def _hist_body(ids_hbm, zeros_hbm, out_hbm, idx_v, vals_v, zbuf_v, hist_sh):
    c = lax.axis_index("c")
    s = lax.axis_index("s")
    wid = s * _NC + c

    pltpu.sync_copy(ids_hbm.at[wid], idx_v)

    for ci in range(_CHUNK // _LANES):
        vals_v[pl.ds(ci * _LANES, _LANES)] = jnp.full((_LANES,), 1.0,
                                                      jnp.float32)

    pltpu.sync_copy(zeros_hbm, zbuf_v)

    @pl.when(s < _NS - 1)
    def _zmain():
        pltpu.sync_copy(zbuf_v, hist_sh.at[pl.ds(s * _ZCH, _ZCH)])

    @pl.when(s == _NS - 1)
    def _zlast():
        pltpu.sync_copy(zbuf_v.at[pl.ds(0, _ZLAST)],
                        hist_sh.at[pl.ds(15 * _ZCH, _ZLAST)])

    plsc.subcore_barrier()

    for k in range(_NCHUNK):
        pltpu.sync_copy(vals_v, hist_sh.at[idx_v.at[k]], add=True)
    plsc.subcore_barrier()

    @pl.when(s < _NS - 1)
    def _dmain():
        pltpu.sync_copy(hist_sh.at[pl.ds(s * _ZCH, _ZCH)], zbuf_v)
        pltpu.sync_copy(zbuf_v, out_hbm.at[c, pl.ds(s * _ZCH, _ZCH)])

    @pl.when(s == _NS - 1)
    def _dlast():
        zpart = zbuf_v.at[pl.ds(0, _ZLAST)]
        pltpu.sync_copy(hist_sh.at[pl.ds(15 * _ZCH, _ZLAST)], zpart)
        pltpu.sync_copy(zpart, out_hbm.at[c, pl.ds(15 * _ZCH, _ZLAST)])


def _scan_body(tbl_ref, cnt_ref, ctail_ref, ttail_ref, w_ref, o_ref, acc_v):
    i = pl.program_id(0)

    @pl.when(i == 0)
    def _init():
        acc_v[...] = jnp.zeros_like(acc_v)

    csum = cnt_ref[0, :] + cnt_ref[1, :]
    acc_v[...] += jnp.dot(
        tbl_ref[...], csum, preferred_element_type=jnp.float32
    )[None, :]

    @pl.when(i == _NMAIN // _C - 1)
    def _finish():
        ct = (ctail_ref[0, :_TAIL] + ctail_ref[1, :_TAIL])
        tail = jnp.dot(ttail_ref[...], ct,
                       preferred_element_type=jnp.float32)
        pooled = (acc_v[0, :] + tail) * (1.0 / _NTOK)
        o_ref[...] = lax.dot_general(
            pooled[None, :], w_ref[...],
            dimension_numbers=(((1,), (1,)), ((), ())),
            preferred_element_type=jnp.float32,
        )


@jax.jit
def _run(ids, emb_t, w):
    mesh = plsc.VectorSubcoreMesh(core_axis_name="c", subcore_axis_name="s")
    counts = pl.kernel(
        _hist_body,
        out_type=jax.ShapeDtypeStruct((_NC, _HPAD), jnp.float32),
        mesh=mesh,
        scratch_types=[
            pltpu.VMEM((_NCHUNK, _CHUNK), jnp.int32),
            pltpu.VMEM((_CHUNK,), jnp.float32),
            pltpu.VMEM((_ZCH,), jnp.float32),
            pltpu.VMEM_SHARED((_HPAD,), jnp.float32),
        ],
        name="token_histogram_sc",
    )(ids, jnp.zeros((_ZCH,), jnp.float32))

    ttail = lax.slice(emb_t, (0, _NMAIN), (_D, _VOCAB))
    tail_blk = _NMAIN // 128
    out = pl.pallas_call(
        _scan_body,
        grid=(_NMAIN // _C,),
        in_specs=[
            pl.BlockSpec((_D, _C), lambda i: (0, i)),
            pl.BlockSpec((_NC, _C), lambda i: (0, i)),
            pl.BlockSpec((_NC, 128), lambda i: (0, tail_blk)),
            pl.BlockSpec((_D, _TAIL), lambda i: (0, 0)),
            pl.BlockSpec((_D, _D), lambda i: (0, 0)),
        ],
        out_specs=pl.BlockSpec((1, _D), lambda i: (0, 0)),
        out_shape=jax.ShapeDtypeStruct((1, _D), jnp.float32),
        scratch_shapes=[pltpu.VMEM((1, _D), jnp.float32)],
        name="table_scan_matvec_tc",
    )(emb_t, counts, counts, ttail, w)
    return out


def kernel(token_ids, embedding, W):
    ids = token_ids.astype(jnp.int32).reshape(_NW, _NCHUNK, _CHUNK)
    return _run(ids, embedding.T, W)


# speedup vs baseline: 1.1091x; 1.0676x over previous
import jax
import jax.numpy as jnp
from jax import lax
from jax.experimental import pallas as pl
from jax.experimental.pallas import tpu as pltpu
from jax.experimental.pallas import tpu_sc as plsc

_NTOK = 16384
_D = 64
_VOCAB = 1000000
_NC = 2
_NS = 16
_NW = _NC * _NS
_PER_W = _NTOK // _NW
_CHUNK = 128
_NCHUNK = _PER_W // _CHUNK
_LANES = 16
_HPAD = 1000064
_VLO1 = 500096
_VSZ0 = _VLO1
_VSZ1 = _HPAD - _VLO1
_TRASH = _VLO1
_HSC = _VLO1 + _CHUNK
_PIDS = _NTOK // _NS
_ICH = _PIDS // _CHUNK
_ZCH = 31232
_ZL0 = _VSZ0 - 15 * _ZCH
_ZL1 = _VSZ1 - 15 * _ZCH
_C = 32256
_NMAIN = 31 * _C
_TAIL = _VOCAB - _NMAIN


def _hist_body(ids_hbm, zeros_hbm, out_hbm, idx_v, vals_v, zbuf_v, hist_sh):
    c = lax.axis_index("c")
    s = lax.axis_index("s")
    lo = c * _VLO1

    pltpu.sync_copy(ids_hbm.at[c, s], idx_v)

    for ci in range(_CHUNK // _LANES):
        vals_v[pl.ds(ci * _LANES, _LANES)] = jnp.full((_LANES,), 1.0,
                                                      jnp.float32)

    pltpu.sync_copy(zeros_hbm, zbuf_v)

    @pl.when(s < _NS - 1)
    def _zmain():
        pltpu.sync_copy(zbuf_v.at[pl.ds(0, _ZCH)],
                        hist_sh.at[pl.ds(s * _ZCH, _ZCH)])

    @pl.when((s == _NS - 1) & (c == 0))
    def _zlast0():
        pltpu.sync_copy(zbuf_v.at[pl.ds(0, _ZL0)],
                        hist_sh.at[pl.ds(15 * _ZCH, _ZL0)])

    @pl.when((s == _NS - 1) & (c == 1))
    def _zlast1():
        pltpu.sync_copy(zbuf_v.at[pl.ds(0, _ZL1)],
                        hist_sh.at[pl.ds(15 * _ZCH, _ZL1)])

    plsc.subcore_barrier()

    for k in range(_ICH):
        pltpu.sync_copy(vals_v, hist_sh.at[idx_v.at[k]], add=True)
    plsc.subcore_barrier()

    @pl.when(s < _NS - 1)
    def _dmain():
        zpart = zbuf_v.at[pl.ds(0, _ZCH)]
        pltpu.sync_copy(hist_sh.at[pl.ds(s * _ZCH, _ZCH)], zpart)
        pltpu.sync_copy(zpart, out_hbm.at[0, pl.ds(lo + s * _ZCH, _ZCH)])

    @pl.when((s == _NS - 1) & (c == 0))
    def _dlast0():
        zpart = zbuf_v.at[pl.ds(0, _ZL0)]
        pltpu.sync_copy(hist_sh.at[pl.ds(15 * _ZCH, _ZL0)], zpart)
        pltpu.sync_copy(zpart, out_hbm.at[0, pl.ds(15 * _ZCH, _ZL0)])

    @pl.when((s == _NS - 1) & (c == 1))
    def _dlast1():
        zpart = zbuf_v.at[pl.ds(0, _ZL1)]
        pltpu.sync_copy(hist_sh.at[pl.ds(15 * _ZCH, _ZL1)], zpart)
        pltpu.sync_copy(
            zpart, out_hbm.at[0, pl.ds(_VLO1 + 15 * _ZCH, _ZL1)])


def _scan_body(tbl_ref, cnt_ref, ctail_ref, ttail_ref, w_ref, o_ref, acc_v):
    i = pl.program_id(0)

    @pl.when(i == 0)
    def _init():
        acc_v[...] = jnp.zeros_like(acc_v)

    acc_v[...] += jnp.dot(
        tbl_ref[...], cnt_ref[0, :], preferred_element_type=jnp.float32
    )[None, :]

    @pl.when(i == _NMAIN // _C - 1)
    def _finish():
        tail = jnp.dot(ttail_ref[...], ctail_ref[0, :_TAIL],
                       preferred_element_type=jnp.float32)
        pooled = (acc_v[0, :] + tail) * (1.0 / _NTOK)
        o_ref[...] = lax.dot_general(
            pooled[None, :], w_ref[...],
            dimension_numbers=(((1,), (1,)), ((), ())),
            preferred_element_type=jnp.float32,
        )


@jax.jit
def _run(ids, emb_t, w):
    mesh = plsc.VectorSubcoreMesh(core_axis_name="c", subcore_axis_name="s")
    counts = pl.kernel(
        _hist_body,
        out_type=jax.ShapeDtypeStruct((1, _HPAD), jnp.float32),
        mesh=mesh,
        scratch_types=[
            pltpu.VMEM((_ICH, _CHUNK), jnp.int32),
            pltpu.VMEM((_CHUNK,), jnp.float32),
            pltpu.VMEM((_ZL0,), jnp.float32),
            pltpu.VMEM_SHARED((_HSC,), jnp.float32),
        ],
        name="token_histogram_sc",
    )(ids, jnp.zeros((_ZL0,), jnp.float32))

    ttail = lax.slice(emb_t, (0, _NMAIN), (_D, _VOCAB))
    tail_blk = _NMAIN // 128
    out = pl.pallas_call(
        _scan_body,
        grid=(_NMAIN // _C,),
        in_specs=[
            pl.BlockSpec((_D, _C), lambda i: (0, i)),
            pl.BlockSpec((1, _C), lambda i: (0, i)),
            pl.BlockSpec((1, 128), lambda i: (0, tail_blk)),
            pl.BlockSpec((_D, _TAIL), lambda i: (0, 0)),
            pl.BlockSpec((_D, _D), lambda i: (0, 0)),
        ],
        out_specs=pl.BlockSpec((1, _D), lambda i: (0, 0)),
        out_shape=jax.ShapeDtypeStruct((1, _D), jnp.float32),
        scratch_shapes=[pltpu.VMEM((1, _D), jnp.float32)],
        name="table_scan_matvec_tc",
    )(emb_t, counts, counts, ttail, w)
    return out


def kernel(token_ids, embedding, W):
    tok = token_ids.astype(jnp.int32)
    lo = jnp.array([0, _VLO1], jnp.int32)[:, None]
    sz = jnp.array([_VSZ0, _VSZ1], jnp.int32)[:, None]
    rel = tok[None, :] - lo
    trash = _TRASH + (jnp.arange(_NTOK, dtype=jnp.int32) % _CHUNK)[None, :]
    ids = jnp.where((rel >= 0) & (rel < sz), rel,
                    trash).reshape(_NC, _NS, _ICH, _CHUNK)
    return _run(ids, embedding.T, W)
